# Initial kernel scaffold; baseline (speedup 1.0000x reference)
import jax
import jax.numpy as jnp
from jax.experimental import pallas as pl


def kernel(h, edge_index, W1, b1, W2, b2, W3, b3):
    raise NotImplementedError("write your pallas kernel here")


# speedup vs baseline: 2.2224x; 2.2224x over previous
import functools

import jax
import jax.numpy as jnp
from jax import lax
from jax.experimental import pallas as pl
from jax.experimental.pallas import tpu as pltpu
from jax.experimental.pallas import tpu_sc as plsc

E = 320000
D = 128
H1 = 256
H2 = 128
CHUNK = 128
NCHUNK = E // CHUNK


def _sc_gather(h, src, dst):
    info = plsc.get_sparse_core_info()
    nc, ns = info.num_cores, info.num_subcores
    nw = nc * ns
    mesh = plsc.VectorSubcoreMesh(core_axis_name="c", subcore_axis_name="s")

    @functools.partial(
        pl.kernel,
        mesh=mesh,
        out_type=(
            jax.ShapeDtypeStruct((E, D), jnp.float32),
            jax.ShapeDtypeStruct((E, D), jnp.float32),
        ),
        scratch_types=[
            pltpu.VMEM((CHUNK,), jnp.int32),
            pltpu.VMEM((CHUNK,), jnp.int32),
            pltpu.VMEM((CHUNK, D), jnp.float32),
            pltpu.VMEM((CHUNK, D), jnp.float32),
            pltpu.SemaphoreType.DMA,
        ],
    )
    def k(h_hbm, src_hbm, dst_hbm, g0_hbm, g1_hbm, idx_s, idx_d, rows_s, rows_d, sem):
        wid = lax.axis_index("s") * nc + lax.axis_index("c")
        base = NCHUNK // nw
        extra = NCHUNK % nw
        nt = base + jnp.where(wid < extra, 1, 0)

        def body(i, carry):
            t = wid + i * nw
            off = t * CHUNK
            pltpu.sync_copy(src_hbm.at[pl.ds(off, CHUNK)], idx_s)
            pltpu.sync_copy(dst_hbm.at[pl.ds(off, CHUNK)], idx_d)
            c1 = pltpu.async_copy(h_hbm.at[idx_s], rows_s, sem)
            c2 = pltpu.async_copy(h_hbm.at[idx_d], rows_d, sem)
            c1.wait()
            c2.wait()
            pltpu.sync_copy(rows_s, g0_hbm.at[pl.ds(off, CHUNK)])
            pltpu.sync_copy(rows_d, g1_hbm.at[pl.ds(off, CHUNK)])
            return carry

        lax.fori_loop(0, nt, body, 0)

    return k(h, src, dst)


R = 1280


def _mlp_body(g0, g1, w1a, w1b, b1, w2, b2, w3, b3, out):
    z = jnp.dot(g0[...], w1a[...], preferred_element_type=jnp.float32)
    z = z + jnp.dot(g1[...], w1b[...], preferred_element_type=jnp.float32)
    z = jnp.maximum(z + b1[...], 0.0)
    z = jnp.dot(z, w2[...], preferred_element_type=jnp.float32) + b2[...]
    z = jnp.maximum(z, 0.0)
    out[...] = jnp.dot(z, w3[...], preferred_element_type=jnp.float32) + b3[...]


def _tc_mlp(g0, g1, w1a, w1b, b1, w2, b2, w3, b3):
    grid = (E // R,)
    return pl.pallas_call(
        _mlp_body,
        grid=grid,
        in_specs=[
            pl.BlockSpec((R, D), lambda i: (i, 0)),
            pl.BlockSpec((R, D), lambda i: (i, 0)),
            pl.BlockSpec((D, H1), lambda i: (0, 0)),
            pl.BlockSpec((D, H1), lambda i: (0, 0)),
            pl.BlockSpec((1, H1), lambda i: (0, 0)),
            pl.BlockSpec((H1, H2), lambda i: (0, 0)),
            pl.BlockSpec((1, H2), lambda i: (0, 0)),
            pl.BlockSpec((H2, 1), lambda i: (0, 0)),
            pl.BlockSpec((1, 1), lambda i: (0, 0)),
        ],
        out_specs=pl.BlockSpec((R, 1), lambda i: (i, 0)),
        out_shape=jax.ShapeDtypeStruct((E, 1), jnp.float32),
    )(g0, g1, w1a, w1b, b1, w2, b2, w3, b3)


def kernel(h, edge_index, W1, b1, W2, b2, W3, b3):
    src = edge_index[0].astype(jnp.int32)
    dst = edge_index[1].astype(jnp.int32)
    g0, g1 = _sc_gather(h, src, dst)
    w1a = W1[:, :D].T
    w1b = W1[:, D:].T
    w2 = W2.T
    w3 = W3.T
    return _tc_mlp(
        g0, g1, w1a, w1b,
        b1.reshape(1, H1), w2, b2.reshape(1, H2), w3, b3.reshape(1, 1),
    )
